# baseline (speedup 1.0000x reference)
import functools

import jax
import jax.numpy as jnp
from jax import lax
from jax.experimental import pallas as pl
from jax.experimental.pallas import tpu as pltpu
from jax.experimental.pallas import tpu_sc as plsc

N = 10000
E = 160000
DIM_IN = 128
H = 128
SCALES = (1.0, 2.0, 3.0, 5.0, 10.0)
RADIUS = 9.0
DIM_IN_TOT = DIM_IN + 2 * len(SCALES)

NC = 2
NS = 16
NW = NC * NS
K = 128
NPAD = 10112
RPT = NPAD // NS
EPAD = 163840
NCHUNKS = EPAD // (NW * K)
KE = 80

def _mesh():
    return plsc.VectorSubcoreMesh(
        core_axis_name="c", subcore_axis_name="s",
        num_cores=NC, num_subcores=NS)


def _iota16():
    return lax.iota(jnp.int32, 16)


def _col(ref, ridx, c):
    return plsc.load_gather(ref, [ridx, jnp.full((16,), c, jnp.int32)])


def _st(ref, ridx, c, val):
    plsc.store_scatter(ref, [ridx, jnp.full((16,), c, jnp.int32)], val)


def _zero_cols(ov, cols, kp):
    if not cols:
        return
    def grp(g, _):
        ridx = g * 16 + _iota16()
        z = jnp.zeros((16,), jnp.float32)
        for c in cols:
            _st(ov, ridx, c, z)
        return 0
    lax.fori_loop(0, kp // 16, grp, 0)


def _sc_edge_pass(name, wt, wo, compute, zero_out_cols, out_kind,
                  second_kind="gather", has_w1b=False, kp=K, wtd=None,
                  inplace=False):
    w2 = wo if second_kind == "linear" else (wtd or wt)
    nch = EPAD // (NW * kp)
    assert nch % 4 == 0
    scratch = [
        pltpu.VMEM((4, kp), jnp.int32),
        pltpu.VMEM((4, kp), jnp.int32),
        pltpu.VMEM((2, kp, wt), jnp.float32),
        pltpu.VMEM((2, kp, w2), jnp.float32),
    ]
    if not inplace:
        scratch.append(pltpu.VMEM((2, kp, wo), jnp.float32))
    if has_w1b:
        scratch.append(pltpu.VMEM((32,), jnp.float32))
    scratch += [pltpu.SemaphoreType.DMA] * 8
    if out_kind == "acc":
        scratch.append(pltpu.VMEM_SHARED((NPAD, wo), jnp.float32))
        out_type = jax.ShapeDtypeStruct((NC, NPAD, wo), jnp.float32)
    else:
        out_type = jax.ShapeDtypeStruct((EPAD, wo), jnp.float32)

    @functools.partial(
        pl.kernel,
        out_type=out_type,
        mesh=_mesh(),
        compiler_params=pltpu.CompilerParams(
            needs_layout_passes=False, use_tc_tiling_on_sc=False),
        scratch_types=scratch,
        name=name,
    )
    def f(*refs):
        n_in = 3 + (1 if second_kind == "linear" else 0) \
            + (1 if out_kind == "acc" else 0) + (1 if has_w1b else 0) \
            + (1 if wtd else 0)
        ins, (out,), scr = refs[:n_in], refs[n_in:n_in + 1], refs[n_in + 1:]
        it = iter(ins)
        tbl = next(it)
        tbld = next(it) if wtd else tbl
        lin = next(it) if second_kind == "linear" else None
        srcp = next(it)
        dstp = next(it)
        zeros = next(it) if out_kind == "acc" else None
        w1b = next(it) if has_w1b else None
        it = iter(scr)
        idxs, idxd, rs, rd = (next(it) for _ in range(4))
        ov = rd if inplace else next(it)
        wv = next(it) if has_w1b else None
        semI = [next(it) for _ in range(4)]
        semG = [next(it) for _ in range(2)]
        semS = [next(it) for _ in range(2)]
        acc = next(it) if out_kind == "acc" else None

        cid = lax.axis_index("c")
        sid = lax.axis_index("s")
        wid = sid * NC + cid
        aux = None
        if has_w1b:
            pltpu.sync_copy(w1b, wv)
            aux = (wv[pl.ds(0, 16)], wv[pl.ds(16, 16)])
        if out_kind == "acc":
            r0 = sid * RPT
            pltpu.sync_copy(zeros.at[pl.ds(r0, RPT)], acc.at[pl.ds(r0, RPT)])
            plsc.subcore_barrier()
        for b in range(2):
            _zero_cols(ov.at[b], zero_out_cols, kp)

        def cbase(c):
            return (c * NW + wid) * kp

        def issue_idx(c, j):
            pltpu.async_copy(srcp.at[pl.ds(cbase(c), kp)], idxs.at[j], semI[j])
            pltpu.async_copy(dstp.at[pl.ds(cbase(c), kp)], idxd.at[j], semI[j])

        def wait_idx(j):
            pltpu.make_async_copy(srcp.at[pl.ds(0, kp)], idxs.at[j],
                                  semI[j]).wait()
            pltpu.make_async_copy(dstp.at[pl.ds(0, kp)], idxd.at[j],
                                  semI[j]).wait()

        def issue_gather(c, j, b):
            pltpu.async_copy(tbl.at[idxs.at[j]], rs.at[b], semG[b])
            if second_kind == "linear":
                pltpu.async_copy(lin.at[pl.ds(cbase(c), kp)], rd.at[b],
                                 semG[b])
            else:
                pltpu.async_copy(tbld.at[idxd.at[j]], rd.at[b], semG[b])

        def wait_gather(j, b):
            pltpu.make_async_copy(tbl.at[idxs.at[j]], rs.at[b],
                                  semG[b]).wait()
            if second_kind == "linear":
                pltpu.make_async_copy(lin.at[pl.ds(0, kp)], rd.at[b],
                                      semG[b]).wait()
            else:
                pltpu.make_async_copy(tbld.at[idxd.at[j]], rd.at[b],
                                      semG[b]).wait()

        def issue_out(c, j, b):
            if out_kind == "acc":
                pltpu.async_copy(ov.at[b], acc.at[idxd.at[j]], semS[b],
                                 add=True)
            else:
                pltpu.async_copy(ov.at[b], out.at[pl.ds(cbase(c), kp)],
                                 semS[b])

        def wait_out(c, j, b):
            if out_kind == "acc":
                pltpu.make_async_copy(ov.at[b], acc.at[idxd.at[j]],
                                      semS[b]).wait()
            else:
                pltpu.make_async_copy(ov.at[b], out.at[pl.ds(cbase(c), kp)],
                                      semS[b]).wait()

        issue_idx(0, 0)
        issue_idx(1, 1)
        wait_idx(0)
        issue_gather(0, 0, 0)

        def body(tt, _):
            for j in range(4):
                b = j % 2
                c = tt * 4 + j

                if not inplace:
                    @pl.when(c >= 2)
                    def _(j=j, b=b, c=c):
                        wait_out(c - 2, (j + 2) % 4, b)

                wait_gather(j, b)
                compute(rs.at[b], rd.at[b], ov.at[b], aux)
                issue_out(c, j, b)

                @pl.when(c + 1 < nch)
                def _(j=j, b=b, c=c):
                    wait_idx((j + 1) % 4)
                    if inplace:
                        @pl.when(c >= 1)
                        def _(j=j, b=b, c=c):
                            wait_out(c - 1, (j + 3) % 4, 1 - b)
                    issue_gather(c + 1, (j + 1) % 4, 1 - b)

                @pl.when(c + 2 < nch)
                def _(j=j, c=c):
                    issue_idx(c + 2, (j + 2) % 4)
            return 0

        lax.fori_loop(0, nch // 4, body, 0)
        if inplace:
            wait_out(nch - 1, (nch - 1) % 4, (nch - 1) % 2)
        else:
            wait_out(nch - 2, (nch - 2) % 4, 0)
            wait_out(nch - 1, (nch - 1) % 4, 1)

        if out_kind == "acc":
            plsc.subcore_barrier()
            pltpu.sync_copy(acc.at[pl.ds(r0, RPT)],
                            out.at[cid, pl.ds(r0, RPT)])

    return f


def _computeA(rs, rd, ov, aux):
    def grp(g, _):
        ridx = g * 16 + _iota16()
        dx = [_col(rs, ridx, k) - _col(rd, ridx, k) for k in range(3)]
        d2 = dx[0] * dx[0] + dx[1] * dx[1] + dx[2] * dx[2]
        n0 = [_col(rs, ridx, 3 + k) for k in range(3)]
        for si, s in enumerate(SCALES):
            w = jnp.exp(d2 * (-1.0 / (2.0 * s * s)))
            for k in range(3):
                _st(ov, ridx, 3 * si + k, w * n0[k])
        return 0
    lax.fori_loop(0, rs.shape[0] // 16, grp, 0)


def _computeB(rs, rd, ov, aux):
    def grp(g, _):
        ridx = g * 16 + _iota16()
        dx = [_col(rs, ridx, k) - _col(rd, ridx, k) for k in range(3)]
        d2 = dx[0] * dx[0] + dx[1] * dx[1] + dx[2] * dx[2]
        for si, s in enumerate(SCALES):
            w = jnp.exp(d2 * (-1.0 / (2.0 * s * s)))
            ns = [_col(rs, ridx, 3 + 3 * si + k) for k in range(3)]
            u = [_col(rd, ridx, 3 + 3 * si + k) for k in range(3)]
            v = [_col(rd, ridx, 18 + 3 * si + k) for k in range(3)]
            p0 = dx[0] * u[0] + dx[1] * u[1] + dx[2] * u[2]
            p1 = dx[0] * v[0] + dx[1] * v[1] + dx[2] * v[2]
            q0 = ns[0] * u[0] + ns[1] * u[1] + ns[2] * u[2]
            q1 = ns[0] * v[0] + ns[1] * v[1] + ns[2] * v[2]
            pq = (p0, p1, q0, q1)
            for r, pr in enumerate((p0, p1)):
                wpr = w * pr
                for c4 in range(4):
                    _st(ov, ridx, 8 * si + r * 4 + c4, wpr * pq[c4])
        return 0
    lax.fori_loop(0, rs.shape[0] // 16, grp, 0)


def _computeC(rs, rd, ov, aux):
    def grp(g, _):
        ridx = g * 16 + _iota16()
        dx = [_col(rs, ridx, k) - _col(rd, ridx, k) for k in range(3)]
        d2 = dx[0] * dx[0] + dx[1] * dx[1] + dx[2] * dx[2]
        coef = jnp.exp(-d2) * _col(rs, ridx, 3)
        for k in range(3):
            _st(ov, ridx, k, coef * dx[k])
        return 0
    lax.fori_loop(0, rs.shape[0] // 16, grp, 0)


def _computeG(rs, rd, ov, aux):
    wa, wb = aux

    def _w(i):
        return wa[i] if i < 16 else wb[i - 16]

    def grp(g, _):
        ridx = g * 16 + _iota16()
        dx = [_col(rs, ridx, k) - _col(rd, ridx, k) for k in range(3)]
        d2 = dx[0] * dx[0] + dx[1] * dx[1] + dx[2] * dx[2]
        nj = [_col(rs, ridx, 3 + k) for k in range(3)]
        ni = [_col(rd, ridx, 3 + k) for k in range(3)]
        dotn = ni[0] * nj[0] + ni[1] * nj[1] + ni[2] * nj[2]
        t = 2.0 - dotn
        win = jnp.exp(-d2 * t * t)
        tu = [_col(rd, ridx, 6 + k) for k in range(3)]
        tv = [_col(rd, ridx, 9 + k) for k in range(3)]
        x0 = dx[0] * tu[0] + dx[1] * tu[1] + dx[2] * tu[2]
        x1 = dx[0] * tv[0] + dx[1] * tv[1] + dx[2] * tv[2]
        x2 = dx[0] * ni[0] + dx[1] * ni[1] + dx[2] * ni[2]
        for j in range(8):
            gj = (x0 * _w(3 * j) + x1 * _w(3 * j + 1)
                  + x2 * _w(3 * j + 2) + _w(24 + j))
            gj = jnp.maximum(gj, 0.0)
            _st(ov, ridx, j, win * gj)
        _st(ov, ridx, 8, win)
        return 0

    lax.fori_loop(0, rs.shape[0] // 16, grp, 0)


def _computeE(rs, rd, ov, aux):
    def erow(e, _):
        for j in range(H // 16):
            sl = pl.ds(j * 16, 16)
            ov[e, sl] = rd[e, sl] * rs[e, sl]
        return 0
    lax.fori_loop(0, rs.shape[0], erow, 0)


def _leaky(x):
    return jnp.where(x >= 0.0, x, 0.2 * x)


def _tangent(n):
    x, y, z = n[:, 0], n[:, 1], n[:, 2]
    s = jnp.where(z >= 0.0, 1.0, -1.0)
    a = -1.0 / (s + z)
    b = x * y * a
    u = jnp.stack([1.0 + s * x * x * a, s * b, -s * x], axis=-1)
    v = jnp.stack([b, s + y * y * a, -y], axis=-1)
    return u, v


def _group_norm(h, gamma, beta, groups=4, eps=1e-5):
    outs = []
    cpg = h.shape[1] // groups
    for g in range(groups):
        blk = h[:, g * cpg:(g + 1) * cpg]
        m = jnp.mean(blk)
        v = jnp.mean((blk - m) ** 2)
        outs.append((blk - m) / jnp.sqrt(v + eps))
    return jnp.concatenate(outs, axis=1) * gamma + beta


def _tc_call(body, out_shape, *args, interpret=False):
    return pl.pallas_call(body, out_shape=out_shape, interpret=interpret)(*args)


_RB = 400
_NRB = N // _RB


def _tc_prep(verts, vnormals, interpret=False):
    def body(v_ref, n_ref, out_ref):
        v = v_ref[...]
        vn = n_ref[...]
        n0 = vn / (jnp.sqrt(jnp.sum(vn * vn, axis=-1, keepdims=True)) + 1e-12)
        u0, v0 = _tangent(n0)
        out_ref[...] = jnp.concatenate(
            [v, n0, u0, v0, jnp.zeros((v.shape[0], 4), jnp.float32)], axis=1)
    return pl.pallas_call(
        body,
        out_shape=jax.ShapeDtypeStruct((N, 16), jnp.float32),
        grid=(_NRB,),
        in_specs=[pl.BlockSpec((_RB, 3), lambda i: (i, 0)),
                  pl.BlockSpec((_RB, 3), lambda i: (i, 0))],
        out_specs=pl.BlockSpec((_RB, 16), lambda i: (i, 0)),
        interpret=interpret,
    )(verts, vnormals)


def _tc_curv(nsum2, tbl0, interpret=False):
    def body(ns_ref, t0_ref, outs_ref, outd_ref):
        nsum = ns_ref[0] + ns_ref[1]
        t0 = t0_ref[...]
        verts = t0[:, 0:3]
        n0 = t0[:, 3:6]
        nss, us, vs = [], [], []
        for si in range(len(SCALES)):
            tot = nsum[:, 3 * si:3 * si + 3] + n0
            ns = tot / (jnp.sqrt(jnp.sum(tot * tot, -1, keepdims=True)) + 1e-12)
            nss.append(ns)
            u, v = _tangent(ns)
            us.append(u)
            vs.append(v)
        outs_ref[...] = jnp.concatenate(
            [verts] + nss + [jnp.zeros((_RB, 14), jnp.float32)], axis=1)
        outd_ref[...] = jnp.concatenate(
            [verts] + us + vs + [jnp.zeros((_RB, 15), jnp.float32)], axis=1)
    return pl.pallas_call(
        body,
        out_shape=(jax.ShapeDtypeStruct((N, 32), jnp.float32),
                   jax.ShapeDtypeStruct((N, 48), jnp.float32)),
        grid=(_NRB,),
        in_specs=[pl.BlockSpec((2, _RB, 16), lambda i: (0, i, 0)),
                  pl.BlockSpec((_RB, 16), lambda i: (i, 0))],
        out_specs=(pl.BlockSpec((_RB, 32), lambda i: (i, 0)),
                   pl.BlockSpec((_RB, 48), lambda i: (i, 0))),
        interpret=interpret,
    )(nsum2, tbl0)


def _tc_curvsolve(ss2, interpret=False):
    def body(ss_ref, out_ref):
        ss = ss_ref[0] + ss_ref[1]
        feats = []
        for si, s in enumerate(SCALES):
            c0 = 8 * si
            ppt00 = ss[:, c0 + 0] + 0.01
            ppt01 = ss[:, c0 + 1]
            pqt00 = ss[:, c0 + 2]
            pqt01 = ss[:, c0 + 3]
            ppt10 = ss[:, c0 + 4]
            ppt11 = ss[:, c0 + 5] + 0.01
            pqt10 = ss[:, c0 + 6]
            pqt11 = ss[:, c0 + 7]
            det = ppt00 * ppt11 - ppt01 * ppt10
            a = (ppt11 * pqt00 - ppt01 * pqt10) / det
            b = (ppt11 * pqt01 - ppt01 * pqt11) / det
            c = (-ppt10 * pqt00 + ppt00 * pqt10) / det
            d = (-ppt10 * pqt01 + ppt00 * pqt11) / det
            feats.append(jnp.clip(0.5 * (a + d), -1.0, 1.0) * s)
            feats.append(jnp.clip(a * d - b * c, -1.0, 1.0) * s * s)
        out_ref[...] = jnp.stack(feats, axis=-1)
    return pl.pallas_call(
        body,
        out_shape=jax.ShapeDtypeStruct((N, 2 * len(SCALES)), jnp.float32),
        grid=(_NRB,),
        in_specs=[pl.BlockSpec((2, _RB, 48), lambda i: (0, i, 0))],
        out_specs=pl.BlockSpec((_RB, 2 * len(SCALES)), lambda i: (i, 0)),
        interpret=interpret,
    )(ss2)


def _tc_dense1(curv, x, tbl0, ws, interpret=False):
    def body(curv_ref, x_ref, t0_ref, osw1, osb1, osw2, osb2,
             inw1, inb1, inw2, inb2, ng, nb, xf_ref, tbl1_ref, f_ref):
        xf = jnp.concatenate([x_ref[...], curv_ref[...]], axis=1)
        xf_ref[...] = xf
        s1 = _leaky(jnp.dot(xf, osw1[...], preferred_element_type=jnp.float32)
                    + osb1[...])
        scores = (jnp.dot(s1, osw2[...], preferred_element_type=jnp.float32)
                  + osb2[...])
        pts = t0_ref[:, 0:3] * (1.0 / RADIUS)
        tbl1_ref[...] = jnp.concatenate(
            [pts, scores, jnp.zeros((N, 12), jnp.float32)], axis=1)
        f1 = _leaky(jnp.dot(xf, inw1[...], preferred_element_type=jnp.float32)
                    + inb1[...])
        f2 = _leaky(jnp.dot(f1, inw2[...], preferred_element_type=jnp.float32)
                    + inb2[...])
        f_ref[...] = _group_norm(f2, ng[...], nb[...])

    out_shapes = (jax.ShapeDtypeStruct((N, DIM_IN_TOT), jnp.float32),
                  jax.ShapeDtypeStruct((N, 16), jnp.float32),
                  jax.ShapeDtypeStruct((N, H), jnp.float32))
    return _tc_call(body, out_shapes, curv, x, tbl0, *ws, interpret=interpret)


def _tc_dense2(ov2, tbl0, interpret=False):
    def body(ov_ref, t0_ref, out_ref):
        ov = ov_ref[0, :, 0:3] + ov_ref[1, :, 0:3]
        t0 = t0_ref[...]
        pts = t0[:, 0:3] * (1.0 / RADIUS)
        n0 = t0[:, 3:6]
        u0 = t0[:, 6:9]
        v0 = t0[:, 9:12]
        oa = jnp.sum(ov * u0, -1) + 1e-5
        ob = jnp.sum(ov * v0, -1)
        onr = jnp.sqrt(oa * oa + ob * ob + 1e-12)
        oa = (oa / onr)[:, None]
        ob = (ob / onr)[:, None]
        tu = oa * u0 + ob * v0
        tv = -ob * u0 + oa * v0
        out_ref[...] = jnp.concatenate(
            [pts, n0, tu, tv, jnp.zeros((_RB, 4), jnp.float32)], axis=1)
    return pl.pallas_call(
        body,
        out_shape=jax.ShapeDtypeStruct((N, 16), jnp.float32),
        grid=(_NRB,),
        in_specs=[pl.BlockSpec((2, _RB, 16), lambda i: (0, i, 0)),
                  pl.BlockSpec((_RB, 16), lambda i: (i, 0))],
        out_specs=pl.BlockSpec((_RB, 16), lambda i: (i, 0)),
        interpret=interpret,
    )(ov2, tbl0)


def _tc_wf(hg, w2t, b2r, interpret=False):
    blk = 4096

    def body(hg_ref, w2_ref, b2_ref, out_ref):
        hgb = hg_ref[...]
        out_ref[...] = (jnp.dot(hgb[:, 0:8], w2_ref[...],
                                preferred_element_type=jnp.float32)
                        + hgb[:, 8:9] * b2_ref[...])

    return pl.pallas_call(
        body,
        out_shape=jax.ShapeDtypeStruct((EPAD, H), jnp.float32),
        grid=(EPAD // blk,),
        in_specs=[
            pl.BlockSpec((blk, 16), lambda i: (i, 0)),
            pl.BlockSpec((8, H), lambda i: (0, 0)),
            pl.BlockSpec((1, H), lambda i: (0, 0)),
        ],
        out_specs=pl.BlockSpec((blk, H), lambda i: (i, 0)),
        interpret=interpret,
    )(hg, w2t, b2r)


def _tc_dense3(agg2, xf, ws, interpret=False):
    def body(agg_ref, xf_ref, ow1, ob1, ow2, ob2, ng, nb,
             lw1, lb1, lw2, lb2, ltw, ltb, out_ref):
        agg = agg_ref[0, :N, :] + agg_ref[1, :N, :]
        xf = xf_ref[...]
        h = _leaky(jnp.dot(agg, ow1[...], preferred_element_type=jnp.float32)
                   + ob1[...])
        h = jnp.dot(h, ow2[...], preferred_element_type=jnp.float32) + ob2[...]
        h = _group_norm(h, ng[...], nb[...])
        h = jnp.maximum(jnp.dot(h, lw1[...], preferred_element_type=jnp.float32)
                        + lb1[...], 0.0)
        h = jnp.dot(h, lw2[...], preferred_element_type=jnp.float32) + lb2[...]
        skip = jnp.dot(xf, ltw[...], preferred_element_type=jnp.float32) + ltb[...]
        out_ref[...] = skip + h
    return _tc_call(body, jax.ShapeDtypeStruct((N, H), jnp.float32),
                    agg2, xf, *ws, interpret=interpret)


@functools.cache
def _sc_impls():
    pa = _sc_edge_pass("sc_nsum", 16, 16, _computeA, [15], "acc", kp=320)
    pb = _sc_edge_pass("sc_sij", 32, 48, _computeB, list(range(40, 48)),
                       "acc", wtd=48, kp=320)
    pc = _sc_edge_pass("sc_orient", 16, 16, _computeC, list(range(3, 16)),
                       "acc", kp=320)
    pc2 = _sc_edge_pass("sc_geom", 16, 16, _computeG, list(range(9, 16)),
                        "linear", has_w1b=True, kp=320)
    pe = _sc_edge_pass("sc_conv", H, H, _computeE, [], "acc",
                       second_kind="linear", kp=KE, inplace=True)
    return dict(
        pA=lambda tbl, s, d: pa(tbl, s, d, jnp.zeros((NPAD, 16), jnp.float32)),
        pB=lambda ts, td, s, d: pb(ts, td, s, d,
                                   jnp.zeros((NPAD, 48), jnp.float32)),
        pC=lambda tbl, s, d: pc(tbl, s, d, jnp.zeros((NPAD, 16), jnp.float32)),
        pC2=lambda te, s, d, w1b: pc2(te, s, d, w1b),
        pE=lambda f, wf, s, d: pe(f, wf, s, d,
                                  jnp.zeros((NPAD, H), jnp.float32)),
    )


def _pad_rows(a):
    return jnp.concatenate(
        [a, jnp.zeros((NPAD - N, a.shape[1]), a.dtype)], axis=0)


def _pipeline(x, verts, vnormals, edge_index, p, impls, interpret=False):
    src = edge_index[0]
    dst = edge_index[1]
    srcp = jnp.concatenate([src, jnp.zeros((EPAD - E,), jnp.int32)])
    dstp = jnp.concatenate([dst, jnp.full((EPAD - E,), N, jnp.int32)])

    tbl0 = _tc_prep(verts, vnormals, interpret=interpret)
    nsum2 = impls["pA"](_pad_rows(tbl0), srcp, dstp)
    tbs, tbd = _tc_curv(nsum2, tbl0, interpret=interpret)
    ss2 = impls["pB"](_pad_rows(tbs), _pad_rows(tbd), srcp, dstp)

    ws1 = (p["os_W1"].T, p["os_b1"], p["os_W2"].T, p["os_b2"],
           p["in_W1"].T, p["in_b1"], p["in_W2"].T, p["in_b2"],
           p["norm_in_g"], p["norm_in_b"])
    curv = _tc_curvsolve(ss2, interpret=interpret)
    xf, tbl1, f = _tc_dense1(curv, x, tbl0, ws1, interpret=interpret)

    ov2 = impls["pC"](_pad_rows(tbl1), srcp, dstp)
    te = _tc_dense2(ov2, tbl0, interpret=interpret)

    w1b = jnp.concatenate([p["cv_W1"].reshape(-1), p["cv_b1"]])
    hg = impls["pC2"](_pad_rows(te), srcp, dstp, w1b)
    wf = _tc_wf(hg, p["cv_W2"].T, p["cv_b2"][None, :], interpret=interpret)

    agg2 = impls["pE"](f, wf, srcp, dstp)

    ws3 = (p["out_W1"].T, p["out_b1"], p["out_W2"].T, p["out_b2"],
           p["norm_out_g"], p["norm_out_b"],
           p["ll_W1"].T, p["ll_b1"], p["ll_W2"].T, p["ll_b2"],
           p["lt_W"].T, p["lt_b"])
    return _tc_dense3(agg2, xf, ws3, interpret=interpret)


def kernel(x, verts, vnormals, edge_index, params):
    return _pipeline(x, verts, vnormals, edge_index, params, _sc_impls())


# speedup vs baseline: 1.0083x; 1.0083x over previous
import functools

import jax
import jax.numpy as jnp
from jax import lax
from jax.experimental import pallas as pl
from jax.experimental.pallas import tpu as pltpu
from jax.experimental.pallas import tpu_sc as plsc

N = 10000
E = 160000
DIM_IN = 128
H = 128
SCALES = (1.0, 2.0, 3.0, 5.0, 10.0)
RADIUS = 9.0
DIM_IN_TOT = DIM_IN + 2 * len(SCALES)

NC = 2
NS = 16
NW = NC * NS
K = 128
NPAD = 10112
RPT = NPAD // NS
EPAD = 163840
NCHUNKS = EPAD // (NW * K)
KE = 80

def _mesh():
    return plsc.VectorSubcoreMesh(
        core_axis_name="c", subcore_axis_name="s",
        num_cores=NC, num_subcores=NS)


def _iota16():
    return lax.iota(jnp.int32, 16)


def _col(ref, ridx, c):
    return plsc.load_gather(ref, [ridx, jnp.full((16,), c, jnp.int32)])


def _st(ref, ridx, c, val):
    plsc.store_scatter(ref, [ridx, jnp.full((16,), c, jnp.int32)], val)


def _zero_cols(ov, cols, kp):
    if not cols:
        return
    def grp(g, _):
        ridx = g * 16 + _iota16()
        z = jnp.zeros((16,), jnp.float32)
        for c in cols:
            _st(ov, ridx, c, z)
        return 0
    lax.fori_loop(0, kp // 16, grp, 0)


def _sc_edge_pass(name, wt, wo, compute, zero_out_cols, out_kind,
                  second_kind="gather", has_w1b=False, kp=K, wtd=None,
                  inplace=False):
    w2 = wo if second_kind == "linear" else (wtd or wt)
    nch = EPAD // (NW * kp)
    assert nch % 4 == 0
    scratch = [
        pltpu.VMEM((4, kp), jnp.int32),
        pltpu.VMEM((4, kp), jnp.int32),
        pltpu.VMEM((2, kp, wt), jnp.float32),
        pltpu.VMEM((2, kp, w2), jnp.float32),
    ]
    if not inplace:
        scratch.append(pltpu.VMEM((2, kp, wo), jnp.float32))
    if has_w1b:
        scratch.append(pltpu.VMEM((32,), jnp.float32))
    scratch += [pltpu.SemaphoreType.DMA] * 8
    if out_kind == "acc":
        scratch.append(pltpu.VMEM_SHARED((NPAD, wo), jnp.float32))
        out_type = jax.ShapeDtypeStruct((NC, NPAD, wo), jnp.float32)
    else:
        out_type = jax.ShapeDtypeStruct((EPAD, wo), jnp.float32)

    @functools.partial(
        pl.kernel,
        out_type=out_type,
        mesh=_mesh(),
        compiler_params=pltpu.CompilerParams(
            needs_layout_passes=False, use_tc_tiling_on_sc=False),
        scratch_types=scratch,
        name=name,
    )
    def f(*refs):
        n_in = 3 + (1 if second_kind == "linear" else 0) \
            + (1 if out_kind == "acc" else 0) + (1 if has_w1b else 0) \
            + (1 if wtd else 0)
        ins, (out,), scr = refs[:n_in], refs[n_in:n_in + 1], refs[n_in + 1:]
        it = iter(ins)
        tbl = next(it)
        tbld = next(it) if wtd else tbl
        lin = next(it) if second_kind == "linear" else None
        srcp = next(it)
        dstp = next(it)
        zeros = next(it) if out_kind == "acc" else None
        w1b = next(it) if has_w1b else None
        it = iter(scr)
        idxs, idxd, rs, rd = (next(it) for _ in range(4))
        ov = rd if inplace else next(it)
        wv = next(it) if has_w1b else None
        semI = [next(it) for _ in range(4)]
        semG = [next(it) for _ in range(2)]
        semS = [next(it) for _ in range(2)]
        acc = next(it) if out_kind == "acc" else None

        cid = lax.axis_index("c")
        sid = lax.axis_index("s")
        wid = sid * NC + cid
        aux = None
        if has_w1b:
            pltpu.sync_copy(w1b, wv)
            aux = (wv[pl.ds(0, 16)], wv[pl.ds(16, 16)])
        if out_kind == "acc":
            r0 = sid * RPT
            pltpu.sync_copy(zeros.at[pl.ds(r0, RPT)], acc.at[pl.ds(r0, RPT)])
            plsc.subcore_barrier()
        for b in range(2):
            _zero_cols(ov.at[b], zero_out_cols, kp)

        def cbase(c):
            return (c * NW + wid) * kp

        def issue_idx(c, j):
            pltpu.async_copy(srcp.at[pl.ds(cbase(c), kp)], idxs.at[j], semI[j])
            pltpu.async_copy(dstp.at[pl.ds(cbase(c), kp)], idxd.at[j], semI[j])

        def wait_idx(j):
            pltpu.make_async_copy(srcp.at[pl.ds(0, kp)], idxs.at[j],
                                  semI[j]).wait()
            pltpu.make_async_copy(dstp.at[pl.ds(0, kp)], idxd.at[j],
                                  semI[j]).wait()

        def issue_gather(c, j, b):
            pltpu.async_copy(tbl.at[idxs.at[j]], rs.at[b], semG[b])
            if second_kind == "linear":
                pltpu.async_copy(lin.at[pl.ds(cbase(c), kp)], rd.at[b],
                                 semG[b])
            else:
                pltpu.async_copy(tbld.at[idxd.at[j]], rd.at[b], semG[b])

        def wait_gather(j, b):
            pltpu.make_async_copy(tbl.at[idxs.at[j]], rs.at[b],
                                  semG[b]).wait()
            if second_kind == "linear":
                pltpu.make_async_copy(lin.at[pl.ds(0, kp)], rd.at[b],
                                      semG[b]).wait()
            else:
                pltpu.make_async_copy(tbld.at[idxd.at[j]], rd.at[b],
                                      semG[b]).wait()

        def issue_out(c, j, b):
            if out_kind == "acc":
                pltpu.async_copy(ov.at[b], acc.at[idxd.at[j]], semS[b],
                                 add=True)
            else:
                pltpu.async_copy(ov.at[b], out.at[pl.ds(cbase(c), kp)],
                                 semS[b])

        def wait_out(c, j, b):
            if out_kind == "acc":
                pltpu.make_async_copy(ov.at[b], acc.at[idxd.at[j]],
                                      semS[b]).wait()
            else:
                pltpu.make_async_copy(ov.at[b], out.at[pl.ds(cbase(c), kp)],
                                      semS[b]).wait()

        issue_idx(0, 0)
        issue_idx(1, 1)
        wait_idx(0)
        issue_gather(0, 0, 0)

        def body(tt, _):
            for j in range(4):
                b = j % 2
                c = tt * 4 + j

                if not inplace:
                    @pl.when(c >= 2)
                    def _(j=j, b=b, c=c):
                        wait_out(c - 2, (j + 2) % 4, b)

                wait_gather(j, b)
                compute(rs.at[b], rd.at[b], ov.at[b], aux)
                issue_out(c, j, b)

                @pl.when(c + 1 < nch)
                def _(j=j, b=b, c=c):
                    wait_idx((j + 1) % 4)
                    if inplace:
                        @pl.when(c >= 1)
                        def _(j=j, b=b, c=c):
                            wait_out(c - 1, (j + 3) % 4, 1 - b)
                    issue_gather(c + 1, (j + 1) % 4, 1 - b)

                @pl.when(c + 2 < nch)
                def _(j=j, c=c):
                    issue_idx(c + 2, (j + 2) % 4)
            return 0

        lax.fori_loop(0, nch // 4, body, 0)
        if inplace:
            wait_out(nch - 1, (nch - 1) % 4, (nch - 1) % 2)
        else:
            wait_out(nch - 2, (nch - 2) % 4, 0)
            wait_out(nch - 1, (nch - 1) % 4, 1)

        if out_kind == "acc":
            plsc.subcore_barrier()
            pltpu.sync_copy(acc.at[pl.ds(r0, RPT)],
                            out.at[cid, pl.ds(r0, RPT)])

    return f


def _computeA(rs, rd, ov, aux):
    def grp(g, _):
        ridx = g * 16 + _iota16()
        dx = [_col(rs, ridx, k) - _col(rd, ridx, k) for k in range(3)]
        d2 = dx[0] * dx[0] + dx[1] * dx[1] + dx[2] * dx[2]
        n0 = [_col(rs, ridx, 3 + k) for k in range(3)]
        for si, s in enumerate(SCALES):
            w = jnp.exp(d2 * (-1.0 / (2.0 * s * s)))
            for k in range(3):
                _st(ov, ridx, 3 * si + k, w * n0[k])
        return 0
    lax.fori_loop(0, rs.shape[0] // 16, grp, 0)


def _computeB(rs, rd, ov, aux):
    def grp(g, _):
        ridx = g * 16 + _iota16()
        dx = [_col(rs, ridx, k) - _col(rd, ridx, k) for k in range(3)]
        d2 = dx[0] * dx[0] + dx[1] * dx[1] + dx[2] * dx[2]
        for si, s in enumerate(SCALES):
            w = jnp.exp(d2 * (-1.0 / (2.0 * s * s)))
            ns = [_col(rs, ridx, 3 + 3 * si + k) for k in range(3)]
            u = [_col(rd, ridx, 3 + 3 * si + k) for k in range(3)]
            v = [_col(rd, ridx, 18 + 3 * si + k) for k in range(3)]
            p0 = dx[0] * u[0] + dx[1] * u[1] + dx[2] * u[2]
            p1 = dx[0] * v[0] + dx[1] * v[1] + dx[2] * v[2]
            q0 = ns[0] * u[0] + ns[1] * u[1] + ns[2] * u[2]
            q1 = ns[0] * v[0] + ns[1] * v[1] + ns[2] * v[2]
            pq = (p0, p1, q0, q1)
            for r, pr in enumerate((p0, p1)):
                wpr = w * pr
                for c4 in range(4):
                    _st(ov, ridx, 8 * si + r * 4 + c4, wpr * pq[c4])
        return 0
    lax.fori_loop(0, rs.shape[0] // 16, grp, 0)


def _computeC(rs, rd, ov, aux):
    def grp(g, _):
        ridx = g * 16 + _iota16()
        dx = [_col(rs, ridx, k) - _col(rd, ridx, k) for k in range(3)]
        d2 = dx[0] * dx[0] + dx[1] * dx[1] + dx[2] * dx[2]
        coef = jnp.exp(-d2) * _col(rs, ridx, 3)
        for k in range(3):
            _st(ov, ridx, k, coef * dx[k])
        return 0
    lax.fori_loop(0, rs.shape[0] // 16, grp, 0)


def _computeG(rs, rd, ov, aux):
    wa, wb = aux

    def _w(i):
        return wa[i] if i < 16 else wb[i - 16]

    def grp(g, _):
        ridx = g * 16 + _iota16()
        dx = [_col(rs, ridx, k) - _col(rd, ridx, k) for k in range(3)]
        d2 = dx[0] * dx[0] + dx[1] * dx[1] + dx[2] * dx[2]
        nj = [_col(rs, ridx, 3 + k) for k in range(3)]
        ni = [_col(rd, ridx, 3 + k) for k in range(3)]
        dotn = ni[0] * nj[0] + ni[1] * nj[1] + ni[2] * nj[2]
        t = 2.0 - dotn
        win = jnp.exp(-d2 * t * t)
        tu = [_col(rd, ridx, 6 + k) for k in range(3)]
        tv = [_col(rd, ridx, 9 + k) for k in range(3)]
        x0 = dx[0] * tu[0] + dx[1] * tu[1] + dx[2] * tu[2]
        x1 = dx[0] * tv[0] + dx[1] * tv[1] + dx[2] * tv[2]
        x2 = dx[0] * ni[0] + dx[1] * ni[1] + dx[2] * ni[2]
        for j in range(8):
            gj = (x0 * _w(3 * j) + x1 * _w(3 * j + 1)
                  + x2 * _w(3 * j + 2) + _w(24 + j))
            gj = jnp.maximum(gj, 0.0)
            _st(ov, ridx, j, win * gj)
        _st(ov, ridx, 8, win)
        return 0

    lax.fori_loop(0, rs.shape[0] // 16, grp, 0)


def _computeE(rs, rd, ov, aux):
    def erow(e, _):
        for j in range(H // 16):
            sl = pl.ds(j * 16, 16)
            ov[e, sl] = rd[e, sl] * rs[e, sl]
        return 0
    lax.fori_loop(0, rs.shape[0], erow, 0)


def _leaky(x):
    return jnp.where(x >= 0.0, x, 0.2 * x)


def _tangent(n):
    x, y, z = n[:, 0], n[:, 1], n[:, 2]
    s = jnp.where(z >= 0.0, 1.0, -1.0)
    a = -1.0 / (s + z)
    b = x * y * a
    u = jnp.stack([1.0 + s * x * x * a, s * b, -s * x], axis=-1)
    v = jnp.stack([b, s + y * y * a, -y], axis=-1)
    return u, v


def _group_norm(h, gamma, beta, groups=4, eps=1e-5):
    outs = []
    cpg = h.shape[1] // groups
    for g in range(groups):
        blk = h[:, g * cpg:(g + 1) * cpg]
        m = jnp.mean(blk)
        v = jnp.mean((blk - m) ** 2)
        outs.append((blk - m) / jnp.sqrt(v + eps))
    return jnp.concatenate(outs, axis=1) * gamma + beta


def _tc_call(body, out_shape, *args, interpret=False):
    return pl.pallas_call(body, out_shape=out_shape, interpret=interpret)(*args)


_RB = 400
_NRB = N // _RB


def _tc_prep(verts, vnormals, interpret=False):
    def body(v_ref, n_ref, out_ref):
        v = v_ref[...]
        vn = n_ref[...]
        n0 = vn / (jnp.sqrt(jnp.sum(vn * vn, axis=-1, keepdims=True)) + 1e-12)
        u0, v0 = _tangent(n0)
        out_ref[...] = jnp.concatenate(
            [v, n0, u0, v0, jnp.zeros((v.shape[0], 4), jnp.float32)], axis=1)
    return pl.pallas_call(
        body,
        out_shape=jax.ShapeDtypeStruct((N, 16), jnp.float32),
        grid=(_NRB,),
        in_specs=[pl.BlockSpec((_RB, 3), lambda i: (i, 0)),
                  pl.BlockSpec((_RB, 3), lambda i: (i, 0))],
        out_specs=pl.BlockSpec((_RB, 16), lambda i: (i, 0)),
        interpret=interpret,
    )(verts, vnormals)


def _tc_curv(nsum2, tbl0, interpret=False):
    def body(ns_ref, t0_ref, outs_ref, outd_ref):
        nsum = ns_ref[0] + ns_ref[1]
        t0 = t0_ref[...]
        verts = t0[:, 0:3]
        n0 = t0[:, 3:6]
        nss, us, vs = [], [], []
        for si in range(len(SCALES)):
            tot = nsum[:, 3 * si:3 * si + 3] + n0
            ns = tot / (jnp.sqrt(jnp.sum(tot * tot, -1, keepdims=True)) + 1e-12)
            nss.append(ns)
            u, v = _tangent(ns)
            us.append(u)
            vs.append(v)
        outs_ref[...] = jnp.concatenate(
            [verts] + nss + [jnp.zeros((_RB, 14), jnp.float32)], axis=1)
        outd_ref[...] = jnp.concatenate(
            [verts] + us + vs + [jnp.zeros((_RB, 15), jnp.float32)], axis=1)
    return pl.pallas_call(
        body,
        out_shape=(jax.ShapeDtypeStruct((N, 32), jnp.float32),
                   jax.ShapeDtypeStruct((N, 48), jnp.float32)),
        grid=(_NRB,),
        in_specs=[pl.BlockSpec((2, _RB, 16), lambda i: (0, i, 0)),
                  pl.BlockSpec((_RB, 16), lambda i: (i, 0))],
        out_specs=(pl.BlockSpec((_RB, 32), lambda i: (i, 0)),
                   pl.BlockSpec((_RB, 48), lambda i: (i, 0))),
        interpret=interpret,
    )(nsum2, tbl0)


def _tc_curvsolve(ss2, interpret=False):
    def body(ss_ref, out_ref):
        ss = ss_ref[0] + ss_ref[1]
        feats = []
        for si, s in enumerate(SCALES):
            c0 = 8 * si
            ppt00 = ss[:, c0 + 0] + 0.01
            ppt01 = ss[:, c0 + 1]
            pqt00 = ss[:, c0 + 2]
            pqt01 = ss[:, c0 + 3]
            ppt10 = ss[:, c0 + 4]
            ppt11 = ss[:, c0 + 5] + 0.01
            pqt10 = ss[:, c0 + 6]
            pqt11 = ss[:, c0 + 7]
            det = ppt00 * ppt11 - ppt01 * ppt10
            a = (ppt11 * pqt00 - ppt01 * pqt10) / det
            b = (ppt11 * pqt01 - ppt01 * pqt11) / det
            c = (-ppt10 * pqt00 + ppt00 * pqt10) / det
            d = (-ppt10 * pqt01 + ppt00 * pqt11) / det
            feats.append(jnp.clip(0.5 * (a + d), -1.0, 1.0) * s)
            feats.append(jnp.clip(a * d - b * c, -1.0, 1.0) * s * s)
        out_ref[...] = jnp.stack(feats, axis=-1)
    return pl.pallas_call(
        body,
        out_shape=jax.ShapeDtypeStruct((N, 2 * len(SCALES)), jnp.float32),
        grid=(_NRB,),
        in_specs=[pl.BlockSpec((2, _RB, 48), lambda i: (0, i, 0))],
        out_specs=pl.BlockSpec((_RB, 2 * len(SCALES)), lambda i: (i, 0)),
        interpret=interpret,
    )(ss2)


def _tc_dense1(curv, x, tbl0, ws, interpret=False):
    def body(curv_ref, x_ref, t0_ref, osw1, osb1, osw2, osb2,
             inw1, inb1, inw2, inb2, ng, nb, xf_ref, tbl1_ref, f_ref):
        xf = jnp.concatenate([x_ref[...], curv_ref[...]], axis=1)
        xf_ref[...] = xf
        s1 = _leaky(jnp.dot(xf, osw1[...], preferred_element_type=jnp.float32)
                    + osb1[...])
        scores = (jnp.dot(s1, osw2[...], preferred_element_type=jnp.float32)
                  + osb2[...])
        pts = t0_ref[:, 0:3] * (1.0 / RADIUS)
        tbl1_ref[...] = jnp.concatenate(
            [pts, scores, jnp.zeros((N, 12), jnp.float32)], axis=1)
        f1 = _leaky(jnp.dot(xf, inw1[...], preferred_element_type=jnp.float32)
                    + inb1[...])
        f2 = _leaky(jnp.dot(f1, inw2[...], preferred_element_type=jnp.float32)
                    + inb2[...])
        f_ref[...] = _group_norm(f2, ng[...], nb[...])

    out_shapes = (jax.ShapeDtypeStruct((N, DIM_IN_TOT), jnp.float32),
                  jax.ShapeDtypeStruct((N, 16), jnp.float32),
                  jax.ShapeDtypeStruct((N, H), jnp.float32))
    return _tc_call(body, out_shapes, curv, x, tbl0, *ws, interpret=interpret)


def _tc_dense2(ov2, tbl0, interpret=False):
    def body(ov_ref, t0_ref, out_ref):
        ov = ov_ref[0, :, 0:3] + ov_ref[1, :, 0:3]
        t0 = t0_ref[...]
        pts = t0[:, 0:3] * (1.0 / RADIUS)
        n0 = t0[:, 3:6]
        u0 = t0[:, 6:9]
        v0 = t0[:, 9:12]
        oa = jnp.sum(ov * u0, -1) + 1e-5
        ob = jnp.sum(ov * v0, -1)
        onr = jnp.sqrt(oa * oa + ob * ob + 1e-12)
        oa = (oa / onr)[:, None]
        ob = (ob / onr)[:, None]
        tu = oa * u0 + ob * v0
        tv = -ob * u0 + oa * v0
        out_ref[...] = jnp.concatenate(
            [pts, n0, tu, tv, jnp.zeros((_RB, 4), jnp.float32)], axis=1)
    return pl.pallas_call(
        body,
        out_shape=jax.ShapeDtypeStruct((N, 16), jnp.float32),
        grid=(_NRB,),
        in_specs=[pl.BlockSpec((2, _RB, 16), lambda i: (0, i, 0)),
                  pl.BlockSpec((_RB, 16), lambda i: (i, 0))],
        out_specs=pl.BlockSpec((_RB, 16), lambda i: (i, 0)),
        interpret=interpret,
    )(ov2, tbl0)


def _tc_wf(hg, w2t, b2r, interpret=False):
    blk = 4096

    def body(hg_ref, w2_ref, b2_ref, out_ref):
        hgb = hg_ref[...]
        out_ref[...] = (jnp.dot(hgb[:, 0:8], w2_ref[...],
                                preferred_element_type=jnp.float32)
                        + hgb[:, 8:9] * b2_ref[...])

    return pl.pallas_call(
        body,
        out_shape=jax.ShapeDtypeStruct((EPAD, H), jnp.float32),
        grid=(EPAD // blk,),
        in_specs=[
            pl.BlockSpec((blk, 16), lambda i: (i, 0)),
            pl.BlockSpec((8, H), lambda i: (0, 0)),
            pl.BlockSpec((1, H), lambda i: (0, 0)),
        ],
        out_specs=pl.BlockSpec((blk, H), lambda i: (i, 0)),
        interpret=interpret,
    )(hg, w2t, b2r)


def _tc_dense3(agg2, xf, ws, interpret=False):
    def body(agg_ref, xf_ref, ow1, ob1, ow2, ob2, ng, nb,
             lw1, lb1, lw2, lb2, ltw, ltb, out_ref):
        agg = agg_ref[0, :N, :] + agg_ref[1, :N, :]
        xf = xf_ref[...]
        h = _leaky(jnp.dot(agg, ow1[...], preferred_element_type=jnp.float32)
                   + ob1[...])
        h = jnp.dot(h, ow2[...], preferred_element_type=jnp.float32) + ob2[...]
        h = _group_norm(h, ng[...], nb[...])
        h = jnp.maximum(jnp.dot(h, lw1[...], preferred_element_type=jnp.float32)
                        + lb1[...], 0.0)
        h = jnp.dot(h, lw2[...], preferred_element_type=jnp.float32) + lb2[...]
        skip = jnp.dot(xf, ltw[...], preferred_element_type=jnp.float32) + ltb[...]
        out_ref[...] = skip + h
    return _tc_call(body, jax.ShapeDtypeStruct((N, H), jnp.float32),
                    agg2, xf, *ws, interpret=interpret)


@functools.cache
def _sc_impls():
    pa = _sc_edge_pass("sc_nsum", 16, 16, _computeA, [15], "acc", kp=256)
    pb = _sc_edge_pass("sc_sij", 32, 48, _computeB, list(range(40, 48)),
                       "acc", wtd=48, kp=256)
    pc = _sc_edge_pass("sc_orient", 16, 16, _computeC, list(range(3, 16)),
                       "acc", kp=256)
    pc2 = _sc_edge_pass("sc_geom", 16, 16, _computeG, list(range(9, 16)),
                        "linear", has_w1b=True, kp=256)
    pe = _sc_edge_pass("sc_conv", H, H, _computeE, [], "acc",
                       second_kind="linear", kp=KE, inplace=True)
    return dict(
        pA=lambda tbl, s, d: pa(tbl, s, d, jnp.zeros((NPAD, 16), jnp.float32)),
        pB=lambda ts, td, s, d: pb(ts, td, s, d,
                                   jnp.zeros((NPAD, 48), jnp.float32)),
        pC=lambda tbl, s, d: pc(tbl, s, d, jnp.zeros((NPAD, 16), jnp.float32)),
        pC2=lambda te, s, d, w1b: pc2(te, s, d, w1b),
        pE=lambda f, wf, s, d: pe(f, wf, s, d,
                                  jnp.zeros((NPAD, H), jnp.float32)),
    )


def _pad_rows(a):
    return jnp.concatenate(
        [a, jnp.zeros((NPAD - N, a.shape[1]), a.dtype)], axis=0)


def _pipeline(x, verts, vnormals, edge_index, p, impls, interpret=False):
    src = edge_index[0]
    dst = edge_index[1]
    srcp = jnp.concatenate([src, jnp.zeros((EPAD - E,), jnp.int32)])
    dstp = jnp.concatenate([dst, jnp.full((EPAD - E,), N, jnp.int32)])

    tbl0 = _tc_prep(verts, vnormals, interpret=interpret)
    nsum2 = impls["pA"](_pad_rows(tbl0), srcp, dstp)
    tbs, tbd = _tc_curv(nsum2, tbl0, interpret=interpret)
    ss2 = impls["pB"](_pad_rows(tbs), _pad_rows(tbd), srcp, dstp)

    ws1 = (p["os_W1"].T, p["os_b1"], p["os_W2"].T, p["os_b2"],
           p["in_W1"].T, p["in_b1"], p["in_W2"].T, p["in_b2"],
           p["norm_in_g"], p["norm_in_b"])
    curv = _tc_curvsolve(ss2, interpret=interpret)
    xf, tbl1, f = _tc_dense1(curv, x, tbl0, ws1, interpret=interpret)

    ov2 = impls["pC"](_pad_rows(tbl1), srcp, dstp)
    te = _tc_dense2(ov2, tbl0, interpret=interpret)

    w1b = jnp.concatenate([p["cv_W1"].reshape(-1), p["cv_b1"]])
    hg = impls["pC2"](_pad_rows(te), srcp, dstp, w1b)
    wf = _tc_wf(hg, p["cv_W2"].T, p["cv_b2"][None, :], interpret=interpret)

    agg2 = impls["pE"](f, wf, srcp, dstp)

    ws3 = (p["out_W1"].T, p["out_b1"], p["out_W2"].T, p["out_b2"],
           p["norm_out_g"], p["norm_out_b"],
           p["ll_W1"].T, p["ll_b1"], p["ll_W2"].T, p["ll_b2"],
           p["lt_W"].T, p["lt_b"])
    return _tc_dense3(agg2, xf, ws3, interpret=interpret)


def kernel(x, verts, vnormals, edge_index, params):
    return _pipeline(x, verts, vnormals, edge_index, params, _sc_impls())
